# baseline (speedup 1.0000x reference)
import functools

import jax
import jax.numpy as jnp
from jax.experimental import pallas as pl
from jax.experimental.pallas import tpu as pltpu
from jax.experimental.pallas import tpu_sc as plsc

B = 128
D = 2048
BLK = 256
NBLK = D // BLK


def _matmul_argmax_kernel(x_ref, W_ref, b_ref, g_ref, G_ref, idx_ref,
                          rmax_ref, ridx_ref):
    j = pl.program_id(0)

    @pl.when(j == 0)
    def _():
        rmax_ref[:] = jnp.full((B, 1), -jnp.inf, jnp.float32)
        ridx_ref[:] = jnp.zeros((B, 1), jnp.int32)

    Gb = jnp.dot(x_ref[:], W_ref[:],
                 preferred_element_type=jnp.float32) + b_ref[:]
    G_ref[:] = Gb
    xb = x_ref[:, pl.ds(j * BLK, BLK)]
    fd = 0.5 * (1.0 - 2.0 * xb) * Gb
    t = fd + g_ref[:]
    bmax = jnp.max(t, axis=1, keepdims=True)
    col = jax.lax.broadcasted_iota(jnp.int32, (B, BLK), 1)
    barg = jnp.min(jnp.where(t == bmax, col, BLK), axis=1,
                   keepdims=True) + j * BLK
    upd = bmax > rmax_ref[:]
    rmax_ref[:] = jnp.where(upd, bmax, rmax_ref[:])
    ridx_ref[:] = jnp.where(upd, barg, ridx_ref[:])

    @pl.when(j == NBLK - 1)
    def _():
        idx_ref[:] = ridx_ref[:]


def _sc_gather(W, idx_flat):
    mesh = plsc.VectorSubcoreMesh(core_axis_name="c", subcore_axis_name="s")

    @functools.partial(
        pl.kernel, mesh=mesh,
        out_type=jax.ShapeDtypeStruct((B, D), jnp.float32),
        scratch_types=[
            pltpu.VMEM((8,), jnp.int32),
            pltpu.VMEM((8, D), jnp.float32),
            pltpu.SemaphoreType.DMA,
        ],
    )
    def gather_kernel(W_hbm, idx_hbm, out_hbm, idx_v, rows_v, sem):
        wid = jax.lax.axis_index("s") * 2 + jax.lax.axis_index("c")

        @pl.when(wid < 16)
        def _():
            base = wid * 8
            pltpu.sync_copy(idx_hbm.at[pl.ds(base, 8)], idx_v)
            pltpu.async_copy(W_hbm.at[idx_v], rows_v, sem).wait()
            pltpu.sync_copy(rows_v, out_hbm.at[pl.ds(base, 8)])

    return gather_kernel(W, idx_flat)


def _finish_kernel(x_ref, G_ref, rows_ref, idx_ref, u_ref, out_ref):
    x = x_ref[:]
    G = G_ref[:]
    rows = rows_ref[:]
    idx = idx_ref[:]
    u = u_ref[:]

    s = 1.0 - 2.0 * x
    fd = 0.5 * s * G
    col = jax.lax.broadcasted_iota(jnp.int32, (B, D), 1)
    changes = (col == idx).astype(jnp.float32)

    mf = jnp.max(fd, axis=1, keepdims=True)
    lse_f = mf[:, 0] + jnp.log(jnp.sum(jnp.exp(fd - mf), axis=1))
    fd_i = jnp.sum(changes * fd, axis=1)
    lp_fwd = fd_i - lse_f

    w_ii = jnp.sum(changes * rows, axis=1)
    s_i = jnp.sum(changes * s, axis=1)
    G_i = jnp.sum(changes * G, axis=1)

    Gp = G + s_i[:, None] * rows
    sp = s * (1.0 - 2.0 * changes)
    rd = 0.5 * sp * Gp
    mr = jnp.max(rd, axis=1, keepdims=True)
    lse_r = mr[:, 0] + jnp.log(jnp.sum(jnp.exp(rd - mr), axis=1))
    rd_i = jnp.sum(changes * rd, axis=1)
    lp_rev = rd_i - lse_r

    m_term = s_i * G_i + 0.5 * w_ii
    la = m_term + lp_rev - lp_fwd
    a = (jnp.exp(la) > u[:, 0]).astype(jnp.float32)
    out_ref[:] = x + (a[:, None] * changes) * s


def kernel(x, W, b):
    key = jax.random.key(42)
    ks, ku = jax.random.split(key)
    g = jax.random.gumbel(ks, x.shape, x.dtype)
    u = jax.random.uniform(ku, (x.shape[0],), x.dtype)

    G, idx = pl.pallas_call(
        _matmul_argmax_kernel,
        grid=(NBLK,),
        in_specs=[
            pl.BlockSpec((B, D), lambda j: (0, 0)),
            pl.BlockSpec((D, BLK), lambda j: (0, j)),
            pl.BlockSpec((1, BLK), lambda j: (0, j)),
            pl.BlockSpec((B, BLK), lambda j: (0, j)),
        ],
        out_specs=[
            pl.BlockSpec((B, BLK), lambda j: (0, j)),
            pl.BlockSpec((B, 1), lambda j: (0, 0)),
        ],
        out_shape=[
            jax.ShapeDtypeStruct((B, D), jnp.float32),
            jax.ShapeDtypeStruct((B, 1), jnp.int32),
        ],
        scratch_shapes=[
            pltpu.VMEM((B, 1), jnp.float32),
            pltpu.VMEM((B, 1), jnp.int32),
        ],
    )(x, W, b.reshape(1, D), g)

    rows = _sc_gather(W, idx.reshape(B))

    return pl.pallas_call(
        _finish_kernel,
        out_shape=jax.ShapeDtypeStruct((B, D), jnp.float32),
    )(x, G, rows, idx, u.reshape(B, 1))


# speedup vs baseline: 1.8536x; 1.8536x over previous
import jax
import jax.numpy as jnp
from jax.experimental import pallas as pl
from jax.experimental.pallas import tpu as pltpu

B = 128
D = 2048
BLK = 256
NBLK = D // BLK


def _gwg_kernel(x_ref, W_hbm, b_ref, g_ref, u_ref, out_ref, Wv, sems):
    copies = []
    for j in range(NBLK):
        sl = pl.ds(j * BLK, BLK)
        cp = pltpu.make_async_copy(W_hbm.at[:, sl], Wv.at[:, sl], sems.at[j])
        cp.start()
        copies.append(cp)

    x = x_ref[:]
    g_parts = []
    for j in range(NBLK):
        sl = pl.ds(j * BLK, BLK)
        copies[j].wait()
        g_parts.append(
            jnp.dot(x, Wv[:, sl], preferred_element_type=jnp.float32)
            + b_ref[:, sl])
    G = jnp.concatenate(g_parts, axis=1)

    s = 1.0 - 2.0 * x
    fd = 0.5 * s * G

    t = fd + g_ref[:]
    tmax = jnp.max(t, axis=1, keepdims=True)
    col = jax.lax.broadcasted_iota(jnp.int32, (B, D), 1)
    idx = jnp.min(jnp.where(t == tmax, col, D), axis=1, keepdims=True)
    changes = (col == idx).astype(jnp.float32)

    mf = jnp.max(fd, axis=1, keepdims=True)
    lse_f = mf[:, 0] + jnp.log(jnp.sum(jnp.exp(fd - mf), axis=1))
    fd_i = jnp.sum(changes * fd, axis=1)
    lp_fwd = fd_i - lse_f

    w_row = jnp.dot(changes, Wv[:, :], preferred_element_type=jnp.float32)
    w_ii = jnp.sum(changes * w_row, axis=1)
    s_i = jnp.sum(changes * s, axis=1)
    G_i = jnp.sum(changes * G, axis=1)

    Gp = G + s_i[:, None] * w_row
    sp = s * (1.0 - 2.0 * changes)
    rd = 0.5 * sp * Gp
    mr = jnp.max(rd, axis=1, keepdims=True)
    lse_r = mr[:, 0] + jnp.log(jnp.sum(jnp.exp(rd - mr), axis=1))
    rd_i = jnp.sum(changes * rd, axis=1)
    lp_rev = rd_i - lse_r

    m_term = s_i * G_i + 0.5 * w_ii
    la = m_term + lp_rev - lp_fwd
    a = (jnp.exp(la) > u_ref[:, 0]).astype(jnp.float32)
    out_ref[:] = x + (a[:, None] * changes) * s


def kernel(x, W, b):
    key = jax.random.key(42)
    ks, ku = jax.random.split(key)
    g = jax.random.gumbel(ks, x.shape, x.dtype)
    u = jax.random.uniform(ku, (x.shape[0],), x.dtype)
    return pl.pallas_call(
        _gwg_kernel,
        in_specs=[
            pl.BlockSpec((B, D), lambda: (0, 0)),
            pl.BlockSpec(memory_space=pl.ANY),
            pl.BlockSpec((1, D), lambda: (0, 0)),
            pl.BlockSpec((B, D), lambda: (0, 0)),
            pl.BlockSpec((B, 1), lambda: (0, 0)),
        ],
        out_shape=jax.ShapeDtypeStruct((B, D), jnp.float32),
        scratch_shapes=[
            pltpu.VMEM((D, D), jnp.float32),
            pltpu.SemaphoreType.DMA((NBLK,)),
        ],
    )(x, W, b.reshape(1, D), g, u.reshape(B, 1))


# speedup vs baseline: 2.7156x; 1.4650x over previous
import jax
import jax.numpy as jnp
from jax.experimental import pallas as pl
from jax.experimental.pallas import tpu as pltpu

B = 128
D = 2048
BLK = 256
NBLK = D // BLK

KS0, KS1 = 0x6D3E048F, 0x1022172D
KU0, KU1 = 0x03D7B32D, 0xADD083F4


def _threefry2x32(k1, k2, x0, x1):
    rot0 = (13, 15, 26, 6)
    rot1 = (17, 29, 16, 24)
    k1 = jnp.uint32(k1)
    k2 = jnp.uint32(k2)
    k3 = k1 ^ k2 ^ jnp.uint32(0x1BD11BDA)
    ks = (k1, k2, k3)
    x0 = x0 + k1
    x1 = x1 + k2

    def four_rounds(x0, x1, rots):
        for r in rots:
            x0 = x0 + x1
            x1 = (x1 << jnp.uint32(r)) | (x1 >> jnp.uint32(32 - r))
            x1 = x0 ^ x1
        return x0, x1

    for i, rots in enumerate((rot0, rot1, rot0, rot1, rot0)):
        x0, x1 = four_rounds(x0, x1, rots)
        x0 = x0 + ks[(i + 1) % 3]
        x1 = x1 + ks[(i + 2) % 3] + jnp.uint32(i + 1)
    return x0, x1


def _bits_to_unit_float(bits):
    fb = (bits >> jnp.uint32(9)) | jnp.uint32(0x3F800000)
    return jax.lax.bitcast_convert_type(fb, jnp.float32) - 1.0


def _gwg_kernel(x_ref, W_hbm, b_ref, out_ref, Wv, sems):
    copies = []
    for j in range(NBLK):
        sl = pl.ds(j * BLK, BLK)
        cp = pltpu.make_async_copy(W_hbm.at[:, sl], Wv.at[:, sl], sems.at[j])
        cp.start()
        copies.append(cp)

    tiny = jnp.float32(1.1754944e-38)
    row = jax.lax.broadcasted_iota(jnp.int32, (B, D), 0)
    colD = jax.lax.broadcasted_iota(jnp.int32, (B, D), 1)
    p = (row * D + colD).astype(jnp.uint32)
    o0, o1 = _threefry2x32(KS0, KS1, jnp.zeros_like(p), p)
    floats = _bits_to_unit_float(o0 ^ o1)
    ug = jnp.maximum(tiny, floats * (1.0 - tiny) + tiny)
    g = -jnp.log(-jnp.log(ug))

    pu = jax.lax.broadcasted_iota(jnp.int32, (B, 1), 0).astype(jnp.uint32)
    uo0, uo1 = _threefry2x32(KU0, KU1, jnp.zeros_like(pu), pu)
    u = _bits_to_unit_float(uo0 ^ uo1)

    x = x_ref[:]
    g_parts = []
    for j in range(NBLK):
        sl = pl.ds(j * BLK, BLK)
        copies[j].wait()
        g_parts.append(
            jnp.dot(x, Wv[:, sl], preferred_element_type=jnp.float32)
            + b_ref[:, sl])
    G = jnp.concatenate(g_parts, axis=1)

    s = 1.0 - 2.0 * x
    fd = 0.5 * s * G

    t = fd + g
    tmax = jnp.max(t, axis=1, keepdims=True)
    idx = jnp.min(jnp.where(t == tmax, colD, D), axis=1, keepdims=True)
    changes = (colD == idx).astype(jnp.float32)

    mf = jnp.max(fd, axis=1, keepdims=True)
    lse_f = mf[:, 0] + jnp.log(jnp.sum(jnp.exp(fd - mf), axis=1))
    fd_i = jnp.sum(changes * fd, axis=1)
    lp_fwd = fd_i - lse_f

    w_row = jnp.dot(changes, Wv[:, :], preferred_element_type=jnp.float32)
    w_ii = jnp.sum(changes * w_row, axis=1)
    s_i = jnp.sum(changes * s, axis=1)
    G_i = jnp.sum(changes * G, axis=1)

    Gp = G + s_i[:, None] * w_row
    sp = s * (1.0 - 2.0 * changes)
    rd = 0.5 * sp * Gp
    mr = jnp.max(rd, axis=1, keepdims=True)
    lse_r = mr[:, 0] + jnp.log(jnp.sum(jnp.exp(rd - mr), axis=1))
    rd_i = jnp.sum(changes * rd, axis=1)
    lp_rev = rd_i - lse_r

    m_term = s_i * G_i + 0.5 * w_ii
    la = m_term + lp_rev - lp_fwd
    a = (jnp.exp(la) > u[:, 0]).astype(jnp.float32)
    out_ref[:] = x + (a[:, None] * changes) * s


def kernel(x, W, b):
    return pl.pallas_call(
        _gwg_kernel,
        in_specs=[
            pl.BlockSpec((B, D), lambda: (0, 0)),
            pl.BlockSpec(memory_space=pl.ANY),
            pl.BlockSpec((1, D), lambda: (0, 0)),
        ],
        out_shape=jax.ShapeDtypeStruct((B, D), jnp.float32),
        scratch_shapes=[
            pltpu.VMEM((D, D), jnp.float32),
            pltpu.SemaphoreType.DMA((NBLK,)),
        ],
    )(x, W, b.reshape(1, D))


# speedup vs baseline: 3.9717x; 1.4626x over previous
import jax
import jax.numpy as jnp
from jax.experimental import pallas as pl
from jax.experimental.pallas import tpu as pltpu

B = 128
D = 2048
RBLK = 512
NROW = D // RBLK

KS0, KS1 = 0x6D3E048F, 0x1022172D
KU0, KU1 = 0x03D7B32D, 0xADD083F4


def _threefry2x32(k1, k2, x0, x1):
    rot0 = (13, 15, 26, 6)
    rot1 = (17, 29, 16, 24)
    k1 = jnp.uint32(k1)
    k2 = jnp.uint32(k2)
    k3 = k1 ^ k2 ^ jnp.uint32(0x1BD11BDA)
    ks = (k1, k2, k3)
    x0 = x0 + k1
    x1 = x1 + k2

    def four_rounds(x0, x1, rots):
        for r in rots:
            x0 = x0 + x1
            x1 = (x1 << jnp.uint32(r)) | (x1 >> jnp.uint32(32 - r))
            x1 = x0 ^ x1
        return x0, x1

    for i, rots in enumerate((rot0, rot1, rot0, rot1, rot0)):
        x0, x1 = four_rounds(x0, x1, rots)
        x0 = x0 + ks[(i + 1) % 3]
        x1 = x1 + ks[(i + 2) % 3] + jnp.uint32(i + 1)
    return x0, x1


def _bits_to_unit_float(bits):
    fb = (bits >> jnp.uint32(9)) | jnp.uint32(0x3F800000)
    return jax.lax.bitcast_convert_type(fb, jnp.float32) - 1.0


def _gwg_kernel(x_ref, W_hbm, b_ref, out_ref, Wv, sems):
    copies = []
    for j in range(NROW):
        sl = pl.ds(j * RBLK, RBLK)
        cp = pltpu.make_async_copy(W_hbm.at[sl, :], Wv.at[sl, :], sems.at[j])
        cp.start()
        copies.append(cp)

    tiny = jnp.float32(1.1754944e-38)
    CH = 128
    g_chunks = []
    for c in range(D // CH):
        row = jax.lax.broadcasted_iota(jnp.int32, (B, CH), 0)
        colc = jax.lax.broadcasted_iota(jnp.int32, (B, CH), 1)
        p = (row * D + (colc + c * CH)).astype(jnp.uint32)
        o0, o1 = _threefry2x32(KS0, KS1, jnp.zeros_like(p), p)
        floats = _bits_to_unit_float(o0 ^ o1)
        ug = jnp.maximum(tiny, floats * (1.0 - tiny) + tiny)
        g_chunks.append(-jnp.log(-jnp.log(ug)))
    g = jnp.concatenate(g_chunks, axis=1)

    pu = jax.lax.broadcasted_iota(jnp.int32, (B, 1), 0).astype(jnp.uint32)
    uo0, uo1 = _threefry2x32(KU0, KU1, jnp.zeros_like(pu), pu)
    u = _bits_to_unit_float(uo0 ^ uo1)

    x = x_ref[:]
    G = b_ref[:] * jnp.ones((B, 1), jnp.float32)
    for j in range(NROW):
        sl = pl.ds(j * RBLK, RBLK)
        copies[j].wait()
        G = G + jnp.dot(x[:, j * RBLK:(j + 1) * RBLK], Wv[sl, :],
                        preferred_element_type=jnp.float32)

    s = 1.0 - 2.0 * x
    fd = 0.5 * s * G

    colD = jax.lax.broadcasted_iota(jnp.int32, (B, D), 1)
    t = fd + g
    tmax = jnp.max(t, axis=1, keepdims=True)
    idx = jnp.min(jnp.where(t == tmax, colD, D), axis=1, keepdims=True)
    changes = (colD == idx).astype(jnp.float32)

    mf = jnp.max(fd, axis=1, keepdims=True)
    lse_f = mf[:, 0] + jnp.log(jnp.sum(jnp.exp(fd - mf), axis=1))
    fd_i = jnp.sum(changes * fd, axis=1)
    lp_fwd = fd_i - lse_f

    w_row = jnp.dot(changes, Wv[:, :], preferred_element_type=jnp.float32)
    w_ii = jnp.sum(changes * w_row, axis=1)
    s_i = jnp.sum(changes * s, axis=1)
    G_i = jnp.sum(changes * G, axis=1)

    Gp = G + s_i[:, None] * w_row
    sp = s * (1.0 - 2.0 * changes)
    rd = 0.5 * sp * Gp
    mr = jnp.max(rd, axis=1, keepdims=True)
    lse_r = mr[:, 0] + jnp.log(jnp.sum(jnp.exp(rd - mr), axis=1))
    rd_i = jnp.sum(changes * rd, axis=1)
    lp_rev = rd_i - lse_r

    m_term = s_i * G_i + 0.5 * w_ii
    la = m_term + lp_rev - lp_fwd
    a = (jnp.exp(la) > u[:, 0]).astype(jnp.float32)
    out_ref[:] = x + (a[:, None] * changes) * s


def kernel(x, W, b):
    return pl.pallas_call(
        _gwg_kernel,
        in_specs=[
            pl.BlockSpec((B, D), lambda: (0, 0)),
            pl.BlockSpec(memory_space=pl.ANY),
            pl.BlockSpec((1, D), lambda: (0, 0)),
        ],
        out_shape=jax.ShapeDtypeStruct((B, D), jnp.float32),
        scratch_shapes=[
            pltpu.VMEM((D, D), jnp.float32),
            pltpu.SemaphoreType.DMA((NROW,)),
        ],
    )(x, W, b.reshape(1, D))
